# Initial kernel scaffold; baseline (speedup 1.0000x reference)
import jax
import jax.numpy as jnp
from jax.experimental import pallas as pl


def kernel(protein_pos, protein_atom_feature, ligand_pos, ligand_atom_feature):
    raise NotImplementedError("write your pallas kernel here")


# speedup vs baseline: 15.8290x; 15.8290x over previous
import functools

import jax
import jax.numpy as jnp
from jax.experimental import pallas as pl
from jax.experimental.pallas import tpu as pltpu

K = 16
NP = 16384
NL = 2048
FDIM = 512
ROWS = 128
PBLK = 1024


def _select_body(lig8_ref, pxt_ref, ligf_ref, counts_ref, ligctx_ref):
    i = pl.program_id(0)
    y = lig8_ref[...]
    xt = pxt_ref[...]
    y2 = jnp.sum(y * y, axis=1, keepdims=True)
    x2 = jnp.sum(xt * xt, axis=0, keepdims=True)
    d2 = y2 + x2 - 2.0 * jnp.dot(y, xt, preferred_element_type=jnp.float32)
    d = jnp.maximum(d2, 0.0)
    col = jax.lax.broadcasted_iota(jnp.int32, (ROWS, NP), 1)
    big = jnp.int32(1 << 30)
    for _ in range(K):
        m = jnp.min(d, axis=1, keepdims=True)
        eq = d == m
        idx = jnp.min(jnp.where(eq, col, big), axis=1, keepdims=True)
        d = jnp.where(eq & (col == idx), jnp.inf, d)
    sel = (d == jnp.inf).astype(jnp.float32)
    cpart = jnp.sum(sel, axis=0)
    lpart = jnp.sum(ligf_ref[...], axis=0)

    @pl.when(i == 0)
    def _():
        counts_ref[...] = cpart
        ligctx_ref[...] = lpart

    @pl.when(i != 0)
    def _():
        counts_ref[...] += cpart
        ligctx_ref[...] += lpart


def _reduce_body(counts_ref, pf_ref, out_ref):
    i = pl.program_id(0)
    w = counts_ref[...]
    f = pf_ref[...]
    part = jnp.sum(w[:, None] * f, axis=0)

    @pl.when(i == 0)
    def _():
        out_ref[...] = part

    @pl.when(i != 0)
    def _():
        out_ref[...] += part


@jax.jit
def kernel(protein_pos, protein_atom_feature, ligand_pos, ligand_atom_feature):
    pos8 = jnp.pad(protein_pos, ((0, 0), (0, 5)))
    lig8 = jnp.pad(ligand_pos, ((0, 0), (0, 5)))
    pxt = pos8.T

    nblk = NL // ROWS
    counts, ligctx = pl.pallas_call(
        _select_body,
        grid=(nblk,),
        in_specs=[
            pl.BlockSpec((ROWS, 8), lambda i: (i, 0)),
            pl.BlockSpec((8, NP), lambda i: (0, 0)),
            pl.BlockSpec((ROWS, FDIM), lambda i: (i, 0)),
        ],
        out_specs=[
            pl.BlockSpec((NP,), lambda i: (0,)),
            pl.BlockSpec((FDIM,), lambda i: (0,)),
        ],
        out_shape=[
            jax.ShapeDtypeStruct((NP,), jnp.float32),
            jax.ShapeDtypeStruct((FDIM,), jnp.float32),
        ],
    )(lig8, pxt, ligand_atom_feature)

    psum = pl.pallas_call(
        _reduce_body,
        grid=(NP // PBLK,),
        in_specs=[
            pl.BlockSpec((PBLK,), lambda i: (i,)),
            pl.BlockSpec((PBLK, FDIM), lambda i: (i, 0)),
        ],
        out_specs=pl.BlockSpec((FDIM,), lambda i: (0,)),
        out_shape=jax.ShapeDtypeStruct((FDIM,), jnp.float32),
    )(counts, protein_atom_feature)

    return jnp.concatenate([ligctx, psum * (1.0 / NL)])
